# Initial kernel scaffold; baseline (speedup 1.0000x reference)
import jax
import jax.numpy as jnp
from jax.experimental import pallas as pl


def kernel(x_seq, edge_index, batch, W1, as1, ad1, b1, W2, as2, ad2, b2, Wq, bq, Wk, bk, Wv, bv, Wo, bo, gamma, beta, Wp, bp):
    raise NotImplementedError("write your pallas kernel here")


# speedup vs baseline: 1.6199x; 1.6199x over previous
import functools

import jax
import jax.numpy as jnp
import numpy as np
from jax.experimental import pallas as pl

N = 10000
E = 160000
D = 128
HG = 4
HID = 64
OUT = 64
S = 6
B = 16
HA = 4


def _proj_kernel(x_ref, w_ref, b_ref, a_ref, h_ref, ea_ref, *, relu_in):
    x = x_ref[...]
    if relu_in:
        x = jnp.maximum(x, 0.0)
    h = jnp.dot(x, w_ref[...], preferred_element_type=jnp.float32) + b_ref[...]
    h_ref[...] = h
    ea_ref[...] = jnp.dot(h, a_ref[...], preferred_element_type=jnp.float32)


def _proj(x, w, b, a_mat, relu_in, blk):
    rows = x.shape[0]
    hc = w.shape[1]
    na = a_mat.shape[1]
    grid = rows // blk
    return pl.pallas_call(
        functools.partial(_proj_kernel, relu_in=relu_in),
        grid=(grid,),
        in_specs=[
            pl.BlockSpec((blk, x.shape[1]), lambda i: (i, 0)),
            pl.BlockSpec((x.shape[1], hc), lambda i: (0, 0)),
            pl.BlockSpec((1, hc), lambda i: (0, 0)),
            pl.BlockSpec((hc, na), lambda i: (0, 0)),
        ],
        out_specs=[
            pl.BlockSpec((blk, hc), lambda i: (i, 0)),
            pl.BlockSpec((blk, na), lambda i: (i, 0)),
        ],
        out_shape=[
            jax.ShapeDtypeStruct((rows, hc), jnp.float32),
            jax.ShapeDtypeStruct((rows, na), jnp.float32),
        ],
    )(x, w, b, a_mat)


def _weight_kernel(alpha_ref, hsrc_ref, rep_ref, w_ref):
    a = jnp.dot(alpha_ref[...], rep_ref[...], preferred_element_type=jnp.float32)
    w_ref[...] = a * hsrc_ref[...]


def _weight(alpha, hsrc, rep, blk):
    rows, hc = hsrc.shape
    h = alpha.shape[1]
    grid = rows // blk
    return pl.pallas_call(
        _weight_kernel,
        grid=(grid,),
        in_specs=[
            pl.BlockSpec((blk, h), lambda i: (i, 0)),
            pl.BlockSpec((blk, hc), lambda i: (i, 0)),
            pl.BlockSpec((h, hc), lambda i: (0, 0)),
        ],
        out_specs=pl.BlockSpec((blk, hc), lambda i: (i, 0)),
        out_shape=jax.ShapeDtypeStruct((rows, hc), jnp.float32),
    )(alpha, hsrc, rep)


def _qkv_kernel(x_ref, wq_ref, bq_ref, wk_ref, bk_ref, wv_ref, bv_ref,
                q_ref, k_ref, v_ref):
    x = x_ref[...]
    q_ref[...] = jnp.dot(x, wq_ref[...], preferred_element_type=jnp.float32) + bq_ref[...]
    k_ref[...] = jnp.dot(x, wk_ref[...], preferred_element_type=jnp.float32) + bk_ref[...]
    v_ref[...] = jnp.dot(x, wv_ref[...], preferred_element_type=jnp.float32) + bv_ref[...]


def _tail_kernel(x_ref, aoc_ref, wo_ref, bo_ref, gamma_ref, beta_ref,
                 wp_ref, bp_ref, out_ref):
    ao = jnp.dot(aoc_ref[...], wo_ref[...], preferred_element_type=jnp.float32) + bo_ref[...]
    y = x_ref[...] + ao
    mu = jnp.mean(y, axis=1, keepdims=True)
    var = jnp.mean((y - mu) ** 2, axis=1, keepdims=True)
    y = (y - mu) * jax.lax.rsqrt(var + 1e-5) * gamma_ref[...] + beta_ref[...]
    ii = jax.lax.broadcasted_iota(jnp.int32, (B, B * S), 0)
    jj = jax.lax.broadcasted_iota(jnp.int32, (B, B * S), 1)
    pool = jnp.where(jj // S == ii, 1.0 / S, 0.0)
    ym = jnp.dot(pool, y, preferred_element_type=jnp.float32)
    out_ref[...] = jnp.dot(ym, wp_ref[...], preferred_element_type=jnp.float32) + bp_ref[...]


def kernel(x_seq, edge_index, batch, W1, as1, ad1, b1, W2, as2, ad2, b2,
           Wq, bq, Wk, bk, Wv, bv, Wo, bo, gamma, beta, Wp, bp):
    src, dst = edge_index[0], edge_index[1]
    counts = jnp.clip(
        jax.ops.segment_sum(jnp.ones((N,), jnp.float32), batch, num_segments=B),
        1.0)

    A1 = jnp.concatenate([
        (jnp.eye(HG, dtype=jnp.float32)[:, None, :] * as1[:, :, None]).reshape(HG * HID, HG),
        (jnp.eye(HG, dtype=jnp.float32)[:, None, :] * ad1[:, :, None]).reshape(HG * HID, HG),
    ], axis=1)
    A2 = jnp.stack([as2[0], ad2[0]], axis=1)
    REP1 = jnp.repeat(jnp.eye(HG, dtype=jnp.float32), HID, axis=1)
    REP2 = jnp.ones((1, HID), jnp.float32)

    def seg_sum_t(vals):
        return jax.vmap(
            lambda v: jax.ops.segment_sum(v, dst, num_segments=N))(vals)

    def edge_softmax(e):
        e = jax.nn.leaky_relu(e, 0.2)
        m = jax.vmap(lambda v: jax.ops.segment_max(v, dst, num_segments=N))(e)
        m = jnp.where(jnp.isfinite(m), m, 0.0)
        ex = jnp.exp(e - m[:, dst, :])
        den = jax.vmap(lambda v: jax.ops.segment_sum(v, dst, num_segments=N))(ex)
        return ex / (den[:, dst, :] + 1e-16)

    x_flat = x_seq.reshape(S * N, D)
    h1, ea1 = _proj(x_flat, W1, b1.reshape(1, -1), A1, relu_in=False, blk=2000)
    h1 = h1.reshape(S, N, HG * HID)
    ea1 = ea1.reshape(S, N, 2 * HG)
    e1 = ea1[:, src, :HG] + ea1[:, dst, HG:]
    alpha1 = edge_softmax(e1)
    hsrc1 = h1[:, src, :]
    w1 = _weight(alpha1.reshape(S * E, HG), hsrc1.reshape(S * E, HG * HID),
                 REP1, blk=4000)
    out1 = seg_sum_t(w1.reshape(S, E, HG * HID)) + b1

    h2, ea2 = _proj(out1.reshape(S * N, HG * HID), W2, b2.reshape(1, -1), A2,
                    relu_in=True, blk=2000)
    h2 = h2.reshape(S, N, HID)
    ea2 = ea2.reshape(S, N, 2)
    e2 = ea2[:, src, :1] + ea2[:, dst, 1:]
    alpha2 = edge_softmax(e2)
    hsrc2 = h2[:, src, :]
    w2 = _weight(alpha2.reshape(S * E, 1), hsrc2.reshape(S * E, HID),
                 REP2, blk=4000)
    out2 = seg_sum_t(w2.reshape(S, E, HID)) + b2

    pooled = jax.vmap(
        lambda v: jax.ops.segment_sum(v, batch, num_segments=B))(out2)
    pooled = pooled / counts[None, :, None]
    x96 = pooled.transpose(1, 0, 2).reshape(B * S, HID)

    spec64 = pl.BlockSpec((HID, HID), lambda: (0, 0))
    spec1x = pl.BlockSpec((1, HID), lambda: (0, 0))
    specx = pl.BlockSpec((B * S, HID), lambda: (0, 0))
    q, k, v = pl.pallas_call(
        _qkv_kernel,
        in_specs=[specx, spec64, spec1x, spec64, spec1x, spec64, spec1x],
        out_specs=[specx, specx, specx],
        out_shape=[jax.ShapeDtypeStruct((B * S, HID), jnp.float32)] * 3,
    )(x96, Wq.T, bq.reshape(1, -1), Wk.T, bk.reshape(1, -1),
      Wv.T, bv.reshape(1, -1))

    dh = HID // HA
    qh = q.reshape(B, S, HA, dh).transpose(0, 2, 1, 3)
    kh = k.reshape(B, S, HA, dh).transpose(0, 2, 1, 3)
    vh = v.reshape(B, S, HA, dh).transpose(0, 2, 1, 3)
    att = jax.nn.softmax((qh @ kh.transpose(0, 1, 3, 2)) / float(np.sqrt(dh)),
                         axis=-1)
    aoc = (att @ vh).transpose(0, 2, 1, 3).reshape(B * S, HID)

    specout = pl.BlockSpec((B, OUT), lambda: (0, 0))
    out = pl.pallas_call(
        _tail_kernel,
        in_specs=[specx, specx, spec64, spec1x, spec1x, spec1x,
                  pl.BlockSpec((HID, OUT), lambda: (0, 0)), spec1x],
        out_specs=specout,
        out_shape=jax.ShapeDtypeStruct((B, OUT), jnp.float32),
    )(x96, aoc, Wo.T, bo.reshape(1, -1), gamma.reshape(1, -1),
      beta.reshape(1, -1), Wp, bp.reshape(1, -1))
    return out
